# baseline (speedup 1.0000x reference)
import functools

import jax
import jax.numpy as jnp
from jax import lax
from jax.experimental import pallas as pl
from jax.experimental.pallas import tpu as pltpu
from jax.experimental.pallas import tpu_sc as plsc

NUM_CORES = 2
NUM_SUBCORES = 16
NW = NUM_CORES * NUM_SUBCORES
LANES = 16

SEQ_L = 200
BATCH = 4096
N = SEQ_L * BATCH
D = 64
N_W = N // NW
CHUNK = 128
G = N_W // CHUNK
NBUF = 4
SCALE = 8.0

_mesh = plsc.VectorSubcoreMesh(core_axis_name="c", subcore_axis_name="s")


@functools.partial(
    pl.kernel,
    out_type=jax.ShapeDtypeStruct((N, D), jnp.float32),
    mesh=_mesh,
    scratch_types=[
        pltpu.VMEM((G, CHUNK), jnp.int32),
        [pltpu.VMEM((CHUNK, D), jnp.float32) for _ in range(NBUF)],
        [pltpu.SemaphoreType.DMA for _ in range(NBUF)],
    ],
    compiler_params=pltpu.CompilerParams(use_tc_tiling_on_sc=False),
)
def _emb_lookup(idx_hbm, table_hbm, out_hbm, idx_v, bufs, sems):
    wid = lax.axis_index("s") * NUM_CORES + lax.axis_index("c")
    base = wid * N_W

    pltpu.sync_copy(idx_hbm.at[wid], idx_v)

    def start(g, b):
        pltpu.async_copy(table_hbm.at[idx_v.at[g]], bufs[b], sems[b])

    def finish(g, b):
        pltpu.make_async_copy(table_hbm.at[idx_v.at[g]], bufs[b], sems[b]).wait()

        def row(r, _):
            for c in range(D // LANES):
                sl = (r, pl.ds(c * LANES, LANES))
                bufs[b][sl] = bufs[b][sl] * SCALE
            return 0

        lax.fori_loop(0, CHUNK, row, 0)
        pltpu.sync_copy(bufs[b], out_hbm.at[pl.ds(base + g * CHUNK, CHUNK)])

    for b in range(NBUF):
        start(b, b)

    def group(k, _):
        for b in range(NBUF):
            g = k * NBUF + b
            finish(g, b)
            start(g + NBUF, b)
        return 0

    lax.fori_loop(0, G // NBUF - 1, group, 0)

    k_last = G // NBUF - 1
    for b in range(NBUF):
        finish(k_last * NBUF + b, b)


def kernel(input_, table):
    idx = input_.reshape(NW, G, CHUNK)
    out = _emb_lookup(idx, table)
    return out.reshape(SEQ_L, BATCH, D)
